# baseline (speedup 1.0000x reference)
import functools

import jax
import jax.numpy as jnp
from jax import lax
from jax.experimental import pallas as pl
from jax.experimental.pallas import tpu as pltpu
from jax.experimental.pallas import tpu_sc as plsc

B = 16
S = 1024
D = 2048
NC = 2
NS = 16
NW = NC * NS
S_PER_W = S // NW
C = 8
SUB = S_PER_W // C
TASKS = B * SUB
L = 16
UNROLL = 8
WQ = 2


def _add_wpe_rows(buf, wpe_v, u, r0, nrows):
  for r in range(r0, r0 + nrows):
    row = u * C + r

    @plsc.parallel_loop(0, D // L, unroll=UNROLL)
    def addbody(j, r=r, row=row):
      off = j * L
      plsc.addupdate(buf.at[r, pl.ds(off, L)], wpe_v[row, pl.ds(off, L)])


def _body(idx_hbm, wte_hbm, wpe_hbm, out_hbm, idx_v, wpe_v, buf0, buf1,
          sg0, sg1, so0, so1):
  wid = lax.axis_index("s") * NC + lax.axis_index("c")
  s0 = wid * S_PER_W

  bufs = (buf0, buf1)
  gsems = (sg0, sg1)
  osems = (so0, so1)

  pltpu.sync_copy(idx_hbm.at[wid], idx_v)
  pltpu.sync_copy(wpe_hbm.at[pl.ds(s0, S_PER_W)], wpe_v)

  def out_base(t):
    return (t // SUB) * S + s0 + (t % SUB) * C

  pltpu.async_copy(wte_hbm.at[idx_v.at[0]], buf0, sg0)

  def step(b, _):
    for u in range(SUB):
      t = SUB * b + u
      p = u % 2
      buf, sg, so = bufs[p], gsems[p], osems[p]
      nbuf, nsg, nso = bufs[p ^ 1], gsems[p ^ 1], osems[p ^ 1]

      pltpu.make_async_copy(wte_hbm.at[idx_v.at[t]], buf, sg).wait()
      _add_wpe_rows(buf, wpe_v, u, 0, WQ)

      @pl.when(t + 1 < TASKS)
      def _prefetch():
        @pl.when(t >= 1)
        def _drain():
          pltpu.make_async_copy(
              nbuf, out_hbm.at[pl.ds(out_base(t - 1), C)], nso).wait()

        pltpu.async_copy(wte_hbm.at[idx_v.at[t + 1]], nbuf, nsg)

      pltpu.async_copy(
          buf.at[pl.ds(0, WQ)], out_hbm.at[pl.ds(out_base(t), WQ)], so)

      for h in range(1, C // WQ):
        _add_wpe_rows(buf, wpe_v, u, h * WQ, WQ)
        pltpu.async_copy(
            buf.at[pl.ds(h * WQ, WQ)],
            out_hbm.at[pl.ds(out_base(t) + h * WQ, WQ)], so)
    return _

  lax.fori_loop(0, B, step, 0)

  for t in (TASKS - 2, TASKS - 1):
    p = t % 2
    pltpu.make_async_copy(
        bufs[p], out_hbm.at[pl.ds(out_base(t), C)], osems[p]).wait()


@functools.partial(
    pl.kernel,
    out_type=jax.ShapeDtypeStruct((B * S, D), jnp.float32),
    mesh=plsc.VectorSubcoreMesh(core_axis_name="c", subcore_axis_name="s"),
    scratch_types=[
        pltpu.VMEM((TASKS, C), jnp.int32),
        pltpu.VMEM((S_PER_W, D), jnp.float32),
        pltpu.VMEM((C, D), jnp.float32),
        pltpu.VMEM((C, D), jnp.float32),
        pltpu.SemaphoreType.DMA,
        pltpu.SemaphoreType.DMA,
        pltpu.SemaphoreType.DMA,
        pltpu.SemaphoreType.DMA,
    ],
)
def _embed_kernel(idx_hbm, wte_hbm, wpe_hbm, out_hbm, idx_v, wpe_v, buf0, buf1,
                  sg0, sg1, so0, so1):
  _body(idx_hbm, wte_hbm, wpe_hbm, out_hbm, idx_v, wpe_v, buf0, buf1,
        sg0, sg1, so0, so1)


def kernel(input_ids, wte, wpe):
  ids = input_ids.astype(jnp.int32)
  idx_prep = (
      ids.reshape(B, NW, SUB, C).transpose(1, 0, 2, 3).reshape(NW, TASKS, C)
  )
  out = _embed_kernel(idx_prep, wte, wpe)
  return out.reshape(B, S, D)


# speedup vs baseline: 1.0951x; 1.0951x over previous
import functools

import jax
import jax.numpy as jnp
from jax import lax
from jax.experimental import pallas as pl
from jax.experimental.pallas import tpu as pltpu
from jax.experimental.pallas import tpu_sc as plsc

B = 16
S = 1024
D = 2048
NC = 2
NS = 16
NW = NC * NS
S_PER_W = S // NW
C = 8
SUB = S_PER_W // C
TASKS = B * SUB
NBUF = 3
GROUP = 12
L = 16
UNROLL = 8
WQ = 2


def _add_wpe_rows(buf, wpe_v, u, r0, nrows):
  for r in range(r0, r0 + nrows):
    row = u * C + r

    @plsc.parallel_loop(0, D // L, unroll=UNROLL)
    def addbody(j, r=r, row=row):
      off = j * L
      plsc.addupdate(buf.at[r, pl.ds(off, L)], wpe_v[row, pl.ds(off, L)])


def _body(idx_hbm, wte_hbm, wpe_hbm, out_hbm, idx_v, wpe_v, bufs, gsems, osems):
  wid = lax.axis_index("s") * NC + lax.axis_index("c")
  s0 = wid * S_PER_W

  pltpu.sync_copy(idx_hbm.at[wid], idx_v)
  pltpu.sync_copy(wpe_hbm.at[pl.ds(s0, S_PER_W)], wpe_v)

  def out_base(t):
    return (t // SUB) * S + s0 + (t % SUB) * C

  def do_task(t, u, p, static_edges=False):
    pn = (p + 1) % NBUF
    buf, sg, so = bufs[p], gsems[p], osems[p]
    nbuf, nsg, nso = bufs[pn], gsems[pn], osems[pn]

    def drain():
      pltpu.make_async_copy(
          nbuf, out_hbm.at[pl.ds(out_base(t - 2), C)], nso).wait()

    def prefetch():
      pltpu.async_copy(wte_hbm.at[idx_v.at[t + 1]], nbuf, nsg)

    if static_edges:
      if t + 1 < TASKS:
        if t >= 2:
          drain()
        prefetch()
    else:
      @pl.when(t + 1 < TASKS)
      def _pf():
        pl.when(t >= 2)(drain)
        prefetch()

    pltpu.make_async_copy(wte_hbm.at[idx_v.at[t]], buf, sg).wait()
    for h in range(C // WQ):
      _add_wpe_rows(buf, wpe_v, u, h * WQ, WQ)
      pltpu.async_copy(
          buf.at[pl.ds(h * WQ, WQ)],
          out_hbm.at[pl.ds(out_base(t) + h * WQ, WQ)], so)

  pltpu.async_copy(wte_hbm.at[idx_v.at[0]], bufs[0], gsems[0])

  n_group = (TASKS // GROUP) * GROUP

  def step(i, _):
    for k in range(GROUP):
      do_task(GROUP * i + k, k % SUB, k % NBUF)
    return _

  lax.fori_loop(0, n_group // GROUP, step, 0)

  for t in range(n_group, TASKS):
    do_task(t, t % SUB, t % NBUF, static_edges=True)

  for t in range(TASKS - NBUF, TASKS):
    p = t % NBUF
    pltpu.make_async_copy(
        bufs[p], out_hbm.at[pl.ds(out_base(t), C)], osems[p]).wait()


@functools.partial(
    pl.kernel,
    out_type=jax.ShapeDtypeStruct((B * S, D), jnp.float32),
    mesh=plsc.VectorSubcoreMesh(core_axis_name="c", subcore_axis_name="s"),
    scratch_types=[
        pltpu.VMEM((TASKS, C), jnp.int32),
        pltpu.VMEM((S_PER_W, D), jnp.float32),
        pltpu.VMEM((C, D), jnp.float32),
        pltpu.VMEM((C, D), jnp.float32),
        pltpu.VMEM((C, D), jnp.float32),
        pltpu.SemaphoreType.DMA,
        pltpu.SemaphoreType.DMA,
        pltpu.SemaphoreType.DMA,
        pltpu.SemaphoreType.DMA,
        pltpu.SemaphoreType.DMA,
        pltpu.SemaphoreType.DMA,
    ],
)
def _embed_kernel(idx_hbm, wte_hbm, wpe_hbm, out_hbm, idx_v, wpe_v,
                  b0, b1, b2, g0, g1, g2, o0, o1, o2):
  _body(idx_hbm, wte_hbm, wpe_hbm, out_hbm, idx_v, wpe_v,
        (b0, b1, b2), (g0, g1, g2), (o0, o1, o2))


def kernel(input_ids, wte, wpe):
  ids = input_ids.astype(jnp.int32)
  idx_prep = (
      ids.reshape(B, NW, SUB, C).transpose(1, 0, 2, 3).reshape(NW, TASKS, C)
  )
  out = _embed_kernel(idx_prep, wte, wpe)
  return out.reshape(B, S, D)


# speedup vs baseline: 1.1255x; 1.0278x over previous
import functools

import jax
import jax.numpy as jnp
from jax import lax
from jax.experimental import pallas as pl
from jax.experimental.pallas import tpu as pltpu
from jax.experimental.pallas import tpu_sc as plsc

B = 16
S = 1024
D = 2048
NC = 2
NS = 16
NW = NC * NS
S_PER_W = S // NW
C = 8
SUB = S_PER_W // C
TASKS = B * SUB
L = 16
UNROLL = 8
WQ = 4


def _add_wpe_rows(buf, wpe_v, u, r0, nrows):
  for r in range(r0, r0 + nrows):
    row = u * C + r

    @plsc.parallel_loop(0, D // L, unroll=UNROLL)
    def addbody(j, r=r, row=row):
      off = j * L
      plsc.addupdate(buf.at[r, pl.ds(off, L)], wpe_v[row, pl.ds(off, L)])


def _body(idx_hbm, wte_hbm, wpe_hbm, out_hbm, idx_v, wpe_v, buf0, buf1,
          sg0, sg1, so0, so1):
  wid = lax.axis_index("s") * NC + lax.axis_index("c")
  s0 = wid * S_PER_W

  bufs = (buf0, buf1)
  gsems = (sg0, sg1)
  osems = (so0, so1)

  pltpu.sync_copy(idx_hbm.at[wid], idx_v)
  pltpu.sync_copy(wpe_hbm.at[pl.ds(s0, S_PER_W)], wpe_v)

  def out_base(t):
    return (t // SUB) * S + s0 + (t % SUB) * C

  pltpu.async_copy(wte_hbm.at[idx_v.at[0]], buf0, sg0)

  def step(b, _):
    for u in range(SUB):
      t = SUB * b + u
      p = u % 2
      buf, sg, so = bufs[p], gsems[p], osems[p]
      nbuf, nsg, nso = bufs[p ^ 1], gsems[p ^ 1], osems[p ^ 1]

      @pl.when(t + 1 < TASKS)
      def _prefetch():
        @pl.when(t >= 1)
        def _drain():
          pltpu.make_async_copy(
              nbuf, out_hbm.at[pl.ds(out_base(t - 1), C)], nso).wait()

        pltpu.async_copy(wte_hbm.at[idx_v.at[t + 1]], nbuf, nsg)

      pltpu.make_async_copy(wte_hbm.at[idx_v.at[t]], buf, sg).wait()
      for h in range(C // WQ):
        _add_wpe_rows(buf, wpe_v, u, h * WQ, WQ)
        pltpu.async_copy(
            buf.at[pl.ds(h * WQ, WQ)],
            out_hbm.at[pl.ds(out_base(t) + h * WQ, WQ)], so)
    return _

  lax.fori_loop(0, B, step, 0)

  for t in (TASKS - 2, TASKS - 1):
    p = t % 2
    pltpu.make_async_copy(
        bufs[p], out_hbm.at[pl.ds(out_base(t), C)], osems[p]).wait()


@functools.partial(
    pl.kernel,
    out_type=jax.ShapeDtypeStruct((B * S, D), jnp.float32),
    mesh=plsc.VectorSubcoreMesh(core_axis_name="c", subcore_axis_name="s"),
    scratch_types=[
        pltpu.VMEM((TASKS, C), jnp.int32),
        pltpu.VMEM((S_PER_W, D), jnp.float32),
        pltpu.VMEM((C, D), jnp.float32),
        pltpu.VMEM((C, D), jnp.float32),
        pltpu.SemaphoreType.DMA,
        pltpu.SemaphoreType.DMA,
        pltpu.SemaphoreType.DMA,
        pltpu.SemaphoreType.DMA,
    ],
)
def _embed_kernel(idx_hbm, wte_hbm, wpe_hbm, out_hbm, idx_v, wpe_v, buf0, buf1,
                  sg0, sg1, so0, so1):
  _body(idx_hbm, wte_hbm, wpe_hbm, out_hbm, idx_v, wpe_v, buf0, buf1,
        sg0, sg1, so0, so1)


def kernel(input_ids, wte, wpe):
  ids = input_ids.astype(jnp.int32)
  idx_prep = (
      ids.reshape(B, NW, SUB, C).transpose(1, 0, 2, 3).reshape(NW, TASKS, C)
  )
  out = _embed_kernel(idx_prep, wte, wpe)
  return out.reshape(B, S, D)


# speedup vs baseline: 1.1496x; 1.0214x over previous
import functools

import jax
import jax.numpy as jnp
from jax import lax
from jax.experimental import pallas as pl
from jax.experimental.pallas import tpu as pltpu
from jax.experimental.pallas import tpu_sc as plsc

B = 16
S = 1024
D = 2048
NC = 2
NS = 16
NW = NC * NS
S_PER_W = S // NW
C = 8
SUB = S_PER_W // C
TASKS = B * SUB
L = 16
UNROLL = 8
WQ = 2


def _add_wpe_rows(buf, wpe_v, u, r0, nrows):
  for r in range(r0, r0 + nrows):
    row = u * C + r

    @plsc.parallel_loop(0, D // L, unroll=UNROLL)
    def addbody(j, r=r, row=row):
      off = j * L
      plsc.addupdate(buf.at[r, pl.ds(off, L)], wpe_v[row, pl.ds(off, L)])


def _body(idx_hbm, wte_hbm, wpe_hbm, out_hbm, idx_v, wpe_v, buf0, buf1,
          sg0, sg1, so0, so1):
  wid = lax.axis_index("s") * NC + lax.axis_index("c")
  s0 = wid * S_PER_W

  bufs = (buf0, buf1)
  gsems = (sg0, sg1)
  osems = (so0, so1)

  for bb in range(B):
    pltpu.async_copy(
        idx_hbm.at[pl.ds(bb * S + s0, S_PER_W)], idx_v.at[bb], sg1)
  for bb in range(B):
    pltpu.make_async_copy(
        idx_hbm.at[pl.ds(bb * S + s0, S_PER_W)], idx_v.at[bb], sg1).wait()

  def out_base(t):
    return (t // SUB) * S + s0 + (t % SUB) * C

  pltpu.async_copy(wte_hbm.at[idx_v.at[0, pl.ds(0, C)]], buf0, sg0)
  pltpu.sync_copy(wpe_hbm.at[pl.ds(s0, S_PER_W)], wpe_v)

  def step(b, _):
    for u in range(SUB):
      t = SUB * b + u
      p = u % 2
      buf, sg, so = bufs[p], gsems[p], osems[p]
      nbuf, nsg, nso = bufs[p ^ 1], gsems[p ^ 1], osems[p ^ 1]

      @pl.when(t + 1 < TASKS)
      def _prefetch():
        @pl.when(t >= 1)
        def _drain():
          pltpu.make_async_copy(
              nbuf, out_hbm.at[pl.ds(out_base(t - 1), C)], nso).wait()

        tn = t + 1
        pltpu.async_copy(
            wte_hbm.at[idx_v.at[tn // SUB, pl.ds((tn % SUB) * C, C)]],
            nbuf, nsg)

      pltpu.make_async_copy(
          wte_hbm.at[idx_v.at[t // SUB, pl.ds((t % SUB) * C, C)]],
          buf, sg).wait()
      for h in range(C // WQ):
        _add_wpe_rows(buf, wpe_v, u, h * WQ, WQ)
        pltpu.async_copy(
            buf.at[pl.ds(h * WQ, WQ)],
            out_hbm.at[pl.ds(out_base(t) + h * WQ, WQ)], so)
    return _

  lax.fori_loop(0, B, step, 0)

  for t in (TASKS - 2, TASKS - 1):
    p = t % 2
    pltpu.make_async_copy(
        bufs[p], out_hbm.at[pl.ds(out_base(t), C)], osems[p]).wait()


@functools.partial(
    pl.kernel,
    out_type=jax.ShapeDtypeStruct((B * S, D), jnp.float32),
    mesh=plsc.VectorSubcoreMesh(core_axis_name="c", subcore_axis_name="s"),
    scratch_types=[
        pltpu.VMEM((B, S_PER_W), jnp.int32),
        pltpu.VMEM((S_PER_W, D), jnp.float32),
        pltpu.VMEM((C, D), jnp.float32),
        pltpu.VMEM((C, D), jnp.float32),
        pltpu.SemaphoreType.DMA,
        pltpu.SemaphoreType.DMA,
        pltpu.SemaphoreType.DMA,
        pltpu.SemaphoreType.DMA,
    ],
)
def _embed_kernel(idx_hbm, wte_hbm, wpe_hbm, out_hbm, idx_v, wpe_v, buf0, buf1,
                  sg0, sg1, so0, so1):
  _body(idx_hbm, wte_hbm, wpe_hbm, out_hbm, idx_v, wpe_v, buf0, buf1,
        sg0, sg1, so0, so1)


def kernel(input_ids, wte, wpe):
  ids = input_ids.astype(jnp.int32).reshape(B * S)
  out = _embed_kernel(ids, wte, wpe)
  return out.reshape(B, S, D)
